# baseline (speedup 1.0000x reference)
import functools

import jax
import jax.numpy as jnp
from jax import lax
from jax.experimental import pallas as pl
from jax.experimental.pallas import tpu as pltpu
from jax.experimental.pallas import tpu_sc as plsc

ALPHA = 1.0
EPS = 1e-7
B = 32
N = 8732
C = 21
L = 16
NPAD = 8736
NCHUNK = 6
CH = NPAD // NCHUNK
GC = CH // L
LN2 = 0.6931471805599453


def _softlog(s):
    bits = lax.bitcast_convert_type(s, jnp.int32)
    e = jnp.float32(1.0) * (lax.shift_right_arithmetic(bits, 23) - 127)
    mbits = lax.bitwise_or(lax.bitwise_and(bits, 0x007FFFFF), 0x3F800000)
    m = lax.bitcast_convert_type(mbits, jnp.float32)
    z = (m - 1.0) / (m + 1.0)
    z2 = z * z
    p = z * (2.0 + z2 * (2.0 / 3.0 + z2 * (2.0 / 5.0 + z2 * (2.0 / 7.0 + z2 * (2.0 / 9.0)))))
    return e * LN2 + p


def _f32_key(v):
    b = lax.bitcast_convert_type(v, jnp.uint32)
    neg = lax.shift_right_logical(b, jnp.uint32(31)) > 0
    return jnp.where(neg, ~b, lax.bitwise_xor(b, jnp.uint32(0x80000000)))


def _key_f32(t):
    was_pos = lax.shift_right_logical(t, jnp.uint32(31)) > 0
    b = jnp.where(was_pos, lax.bitwise_xor(t, jnp.uint32(0x80000000)), ~t)
    return lax.bitcast_convert_type(b, jnp.float32)


def _sc_body(conf_hbm, tgt_hbm, loc_hbm, out_hbm,
             conf_v, tgt_v, loc_v, key_v, hist_v, out_v, sem):
    w = lax.axis_index("s") * 2 + lax.axis_index("c")
    ar = lax.iota(jnp.int32, L)
    zero = jnp.zeros((L,), jnp.float32)

    def make_group(start_row, masked):
        def group_body(g, accs):
            pos_acc, nm_acc, loc_acc = accs
            base = g * L

            xs = [conf_v[pl.ds(c * CH + base, L)] for c in range(C)]
            m = xs[0]
            for c in range(1, C):
                m = jnp.maximum(m, xs[c])
            s = zero
            for c in range(C):
                s = s + jnp.exp(xs[c] - m)
            lse = m + _softlog(s)

            lab_f = tgt_v[pl.ds(4 * CH + base, L)]
            lab = lab_f.astype(jnp.int32)
            pos = lab > 0
            safe_lab = jnp.where(pos, jnp.clip(lab, 0, C - 1), 0)
            x_lab = plsc.load_gather(conf_v, [safe_lab * CH + base + ar])

            if masked:
                valid = (start_row + base + ar) < N
                posv = jnp.logical_and(pos, valid)
                drop = jnp.logical_or(pos, jnp.logical_not(valid))
            else:
                posv = pos
                drop = pos
            pos_acc = pos_acc + jnp.where(posv, lse - x_lab, 0.0)
            nm_acc = nm_acc + jnp.where(posv, 1.0, 0.0)

            ngv = jnp.where(drop, jnp.float32(-1e30), lse - xs[0])
            key_v[pl.ds(start_row + base, L)] = _f32_key(ngv)

            x1 = loc_v[pl.ds(0 * CH + base, L)]
            y1 = loc_v[pl.ds(1 * CH + base, L)]
            x2 = loc_v[pl.ds(2 * CH + base, L)]
            y2 = loc_v[pl.ds(3 * CH + base, L)]
            x1g = tgt_v[pl.ds(0 * CH + base, L)]
            y1g = tgt_v[pl.ds(1 * CH + base, L)]
            x2g = tgt_v[pl.ds(2 * CH + base, L)]
            y2g = tgt_v[pl.ds(3 * CH + base, L)]
            xkis1 = jnp.maximum(x1, x1g)
            ykis1 = jnp.maximum(y1, y1g)
            xkis2 = jnp.minimum(x2, x2g)
            ykis2 = jnp.minimum(y2, y2g)
            imask = jnp.logical_and(ykis2 > ykis1, xkis2 > xkis1)
            intsctk = jnp.where(imask, (xkis2 - xkis1) * (ykis2 - ykis1), 0.0)
            unionk = (x2 - x1) * (y2 - y1) + (x2g - x1g) * (y2g - y1g) - intsctk
            iouk = intsctk / (unionk + EPS)
            area_c = (jnp.maximum(x2, x2g) - jnp.minimum(x1, x1g)) * \
                     (jnp.maximum(y2, y2g) - jnp.minimum(y1, y1g))
            miouk = iouk - (area_c - unionk) / (area_c + EPS)
            loc_acc = loc_acc + jnp.where(posv, 1.0 - miouk, 0.0)
            return pos_acc, nm_acc, loc_acc

        return group_body

    def chunk_accs(ci, accs, masked):
        c1 = pltpu.async_copy(
            conf_hbm.at[pl.ds((w * NCHUNK + ci) * (CH * C), CH * C)], conf_v, sem)
        c2 = pltpu.async_copy(
            tgt_hbm.at[pl.ds((w * NCHUNK + ci) * (CH * 5), CH * 5)], tgt_v, sem)
        c3 = pltpu.async_copy(
            loc_hbm.at[pl.ds((w * NCHUNK + ci) * (CH * 4), CH * 4)], loc_v, sem)
        c1.wait()
        c2.wait()
        c3.wait()
        return lax.fori_loop(0, GC, make_group(ci * CH, masked), accs)

    accs = lax.fori_loop(
        0, NCHUNK - 1,
        lambda ci, a: chunk_accs(ci, a, False),
        (zero, zero, zero))
    pos_acc, nm_acc, loc_acc = chunk_accs(NCHUNK - 1, accs, True)

    pos_loss = jnp.sum(pos_acc, axis=0)
    nm_f = jnp.sum(nm_acc, axis=0)
    loc_loss = jnp.sum(loc_acc, axis=0)
    nm = nm_f.astype(jnp.int32)
    k = jnp.minimum(3 * nm, N - nm)
    k_f = k.astype(jnp.float32)

    NG = NPAD // L
    UNROLL = 6
    NGU = NG // UNROLL

    ones = jnp.ones((L,), jnp.float32)

    def clear_hist(nbins):
        def zbody(i, _):
            hist_v[pl.ds(i * L, L)] = jnp.zeros((L,), jnp.float32)
            return 0
        lax.fori_loop(0, nbins, zbody, 0)

    def build_hist(shift, nbits, prefix, pshift, use_mask):
        def hbody(g, _):
            for u in range(UNROLL):
                kv = key_v[pl.ds((g * UNROLL + u) * L, L)]
                binv = lax.bitwise_and(
                    lax.shift_right_logical(kv, jnp.uint32(shift)),
                    jnp.uint32((1 << nbits) - 1)).astype(jnp.int32)
                idx = binv * L + ar
                if use_mask:
                    mk = lax.shift_right_logical(kv, jnp.uint32(pshift)) == prefix
                    plsc.addupdate_scatter(hist_v, [idx], ones, mask=mk)
                else:
                    plsc.addupdate_scatter(hist_v, [idx], ones)
            return 0
        lax.fori_loop(0, NGU, hbody, 0)

    def scan_hist(nbins, need):
        def cond(c):
            _, _, cum_a = c
            return cum_a < need
        def body(c):
            bin_, _, cum_a = c
            nb = bin_ - 1
            h = jnp.sum(hist_v[pl.ds(nb * L, L)], axis=0)
            return (nb, cum_a, cum_a + h)
        bin_, cum_b, _ = lax.while_loop(
            cond, body, (jnp.int32(nbins), jnp.float32(0.0), jnp.float32(0.0)))
        return bin_, cum_b

    need = jnp.maximum(k_f, 1.0)

    clear_hist(512)
    build_hist(23, 9, jnp.uint32(0), 0, False)
    b1, above1 = scan_hist(512, need)
    need = need - above1
    p1 = b1.astype(jnp.uint32)

    clear_hist(256)
    build_hist(15, 8, p1, 23, True)
    b2, above2 = scan_hist(256, need)
    need = need - above2
    p2 = lax.bitwise_or(lax.shift_left(p1, jnp.uint32(8)), b2.astype(jnp.uint32))

    clear_hist(256)
    build_hist(7, 8, p2, 15, True)
    b3, above3 = scan_hist(256, need)
    need = need - above3
    p3 = lax.bitwise_or(lax.shift_left(p2, jnp.uint32(8)), b3.astype(jnp.uint32))

    clear_hist(128)
    build_hist(0, 7, p3, 7, True)
    b4, _ = scan_hist(128, need)
    t = lax.bitwise_or(lax.shift_left(p3, jnp.uint32(7)), b4.astype(jnp.uint32))
    tvec = jnp.full((L,), t)

    def tail_body(g, accs):
        cnt_acc, sum_acc = accs
        for u in range(UNROLL):
            kv = key_v[pl.ds((g * UNROLL + u) * L, L)]
            gt = kv > tvec
            cnt_acc = cnt_acc + jnp.where(gt, 1.0, 0.0)
            sum_acc = sum_acc + jnp.where(gt, _key_f32(kv), 0.0)
        return (cnt_acc, sum_acc)

    cnt_gt, sum_gt = lax.fori_loop(0, NGU, tail_body, (zero, zero))
    cnt_gt = jnp.sum(cnt_gt, axis=0)
    sum_gt = jnp.sum(sum_gt, axis=0)
    neg_loss = jnp.where(k > 0, sum_gt + (k_f - cnt_gt) * _key_f32(t), 0.0)

    total = pos_loss + neg_loss + ALPHA * loc_loss
    out_v[...] = jnp.where(ar == 0, total, jnp.where(ar == 1, nm_f, 0.0))
    pltpu.sync_copy(out_v, out_hbm.at[pl.ds(w * L, L)])


def _tc_combine_body(part_ref, o_ref):
    total = jnp.sum(part_ref[:, 0:1], keepdims=True)
    nh = jnp.sum(part_ref[:, 1:2], keepdims=True)
    o_ref[...] = jnp.where(nh == 0.0, jnp.float32(0.0),
                           total / jnp.maximum(nh, 1.0))


def kernel(confidences, localizations, targets):
    pad = NPAD - N
    conf_p = jnp.pad(confidences, ((0, 0), (0, pad), (0, 0)))
    tgt_p = jnp.pad(targets, ((0, 0), (0, pad), (0, 0)))
    loc_p = jnp.pad(localizations, ((0, 0), (0, pad), (0, 0)))
    conf_b = conf_p.reshape(B, NCHUNK, CH, C).transpose(0, 1, 3, 2).reshape(-1)
    tgt_b = tgt_p.reshape(B, NCHUNK, CH, 5).transpose(0, 1, 3, 2).reshape(-1)
    loc_b = loc_p.reshape(B, NCHUNK, CH, 4).transpose(0, 1, 3, 2).reshape(-1)

    mesh = plsc.VectorSubcoreMesh(core_axis_name="c", subcore_axis_name="s",
                                  num_cores=2, num_subcores=16)
    parts = pl.kernel(
        _sc_body,
        out_type=jax.ShapeDtypeStruct((B * L,), jnp.float32),
        mesh=mesh,
        compiler_params=pltpu.CompilerParams(needs_layout_passes=False),
        scratch_types=[
            pltpu.VMEM((CH * C,), jnp.float32),
            pltpu.VMEM((CH * 5,), jnp.float32),
            pltpu.VMEM((CH * 4,), jnp.float32),
            pltpu.VMEM((NPAD,), jnp.uint32),
            pltpu.VMEM((512 * L,), jnp.float32),
            pltpu.VMEM((L,), jnp.float32),
            pltpu.SemaphoreType.DMA,
        ],
    )(conf_b, tgt_b, loc_b)

    out = pl.pallas_call(
        _tc_combine_body,
        out_shape=jax.ShapeDtypeStruct((1, 1), jnp.float32),
    )(parts.reshape(B, L))
    return out[0, 0]


# speedup vs baseline: 1.0569x; 1.0569x over previous
import functools

import jax
import jax.numpy as jnp
from jax import lax
from jax.experimental import pallas as pl
from jax.experimental.pallas import tpu as pltpu
from jax.experimental.pallas import tpu_sc as plsc

ALPHA = 1.0
EPS = 1e-7
B = 32
N = 8732
C = 21
L = 16
NPAD = 8736
NCHUNK = 6
CH = NPAD // NCHUNK
GC = CH // L
LN2 = 0.6931471805599453


def _softlog(s):
    bits = lax.bitcast_convert_type(s, jnp.int32)
    e = jnp.float32(1.0) * (lax.shift_right_arithmetic(bits, 23) - 127)
    mbits = lax.bitwise_or(lax.bitwise_and(bits, 0x007FFFFF), 0x3F800000)
    m = lax.bitcast_convert_type(mbits, jnp.float32)
    z = (m - 1.0) / (m + 1.0)
    z2 = z * z
    p = z * (2.0 + z2 * (2.0 / 3.0 + z2 * (2.0 / 5.0 + z2 * (2.0 / 7.0 + z2 * (2.0 / 9.0)))))
    return e * LN2 + p


def _f32_key(v):
    b = lax.bitcast_convert_type(v, jnp.uint32)
    neg = lax.shift_right_logical(b, jnp.uint32(31)) > 0
    return jnp.where(neg, ~b, lax.bitwise_xor(b, jnp.uint32(0x80000000)))


def _key_f32(t):
    was_pos = lax.shift_right_logical(t, jnp.uint32(31)) > 0
    b = jnp.where(was_pos, lax.bitwise_xor(t, jnp.uint32(0x80000000)), ~t)
    return lax.bitcast_convert_type(b, jnp.float32)


def _sc_body(conf_hbm, tgt_hbm, loc_hbm, out_hbm,
             conf_v, tgt_v, loc_v, key_v, out_v, sem):
    w = lax.axis_index("s") * 2 + lax.axis_index("c")
    ar = lax.iota(jnp.int32, L)
    zero = jnp.zeros((L,), jnp.float32)

    def make_group(start_row, masked):
        def group_body(g, accs):
            pos_acc, nm_acc, loc_acc = accs
            base = g * L

            xs = [conf_v[pl.ds(c * CH + base, L)] for c in range(C)]
            m = xs[0]
            for c in range(1, C):
                m = jnp.maximum(m, xs[c])
            s = zero
            for c in range(C):
                s = s + jnp.exp(xs[c] - m)
            lse = m + _softlog(s)

            lab_f = tgt_v[pl.ds(4 * CH + base, L)]
            lab = lab_f.astype(jnp.int32)
            pos = lab > 0
            safe_lab = jnp.where(pos, jnp.clip(lab, 0, C - 1), 0)
            x_lab = plsc.load_gather(conf_v, [safe_lab * CH + base + ar])

            if masked:
                valid = (start_row + base + ar) < N
                posv = jnp.logical_and(pos, valid)
                drop = jnp.logical_or(pos, jnp.logical_not(valid))
            else:
                posv = pos
                drop = pos
            pos_acc = pos_acc + jnp.where(posv, lse - x_lab, 0.0)
            nm_acc = nm_acc + jnp.where(posv, 1.0, 0.0)

            ngv = jnp.where(drop, jnp.float32(-1e30), lse - xs[0])
            key_v[pl.ds(start_row + base, L)] = _f32_key(ngv)

            x1 = loc_v[pl.ds(0 * CH + base, L)]
            y1 = loc_v[pl.ds(1 * CH + base, L)]
            x2 = loc_v[pl.ds(2 * CH + base, L)]
            y2 = loc_v[pl.ds(3 * CH + base, L)]
            x1g = tgt_v[pl.ds(0 * CH + base, L)]
            y1g = tgt_v[pl.ds(1 * CH + base, L)]
            x2g = tgt_v[pl.ds(2 * CH + base, L)]
            y2g = tgt_v[pl.ds(3 * CH + base, L)]
            xkis1 = jnp.maximum(x1, x1g)
            ykis1 = jnp.maximum(y1, y1g)
            xkis2 = jnp.minimum(x2, x2g)
            ykis2 = jnp.minimum(y2, y2g)
            imask = jnp.logical_and(ykis2 > ykis1, xkis2 > xkis1)
            intsctk = jnp.where(imask, (xkis2 - xkis1) * (ykis2 - ykis1), 0.0)
            unionk = (x2 - x1) * (y2 - y1) + (x2g - x1g) * (y2g - y1g) - intsctk
            iouk = intsctk / (unionk + EPS)
            area_c = (jnp.maximum(x2, x2g) - jnp.minimum(x1, x1g)) * \
                     (jnp.maximum(y2, y2g) - jnp.minimum(y1, y1g))
            miouk = iouk - (area_c - unionk) / (area_c + EPS)
            loc_acc = loc_acc + jnp.where(posv, 1.0 - miouk, 0.0)
            return pos_acc, nm_acc, loc_acc

        return group_body

    def chunk_accs(ci, accs, masked):
        c1 = pltpu.async_copy(
            conf_hbm.at[pl.ds((w * NCHUNK + ci) * (CH * C), CH * C)], conf_v, sem)
        c2 = pltpu.async_copy(
            tgt_hbm.at[pl.ds((w * NCHUNK + ci) * (CH * 5), CH * 5)], tgt_v, sem)
        c3 = pltpu.async_copy(
            loc_hbm.at[pl.ds((w * NCHUNK + ci) * (CH * 4), CH * 4)], loc_v, sem)
        c1.wait()
        c2.wait()
        c3.wait()
        return lax.fori_loop(0, GC, make_group(ci * CH, masked), accs)

    accs = lax.fori_loop(
        0, NCHUNK - 1,
        lambda ci, a: chunk_accs(ci, a, False),
        (zero, zero, zero))
    pos_acc, nm_acc, loc_acc = chunk_accs(NCHUNK - 1, accs, True)

    pos_loss = jnp.sum(pos_acc, axis=0)
    nm_f = jnp.sum(nm_acc, axis=0)
    loc_loss = jnp.sum(loc_acc, axis=0)
    nm = nm_f.astype(jnp.int32)
    k = jnp.minimum(3 * nm, N - nm)
    k_f = k.astype(jnp.float32)

    NG = NPAD // L
    UNROLL = 13
    NGU = NG // UNROLL

    def search_body(bi, t):
        cand = lax.bitwise_or(t, lax.shift_left(jnp.uint32(1), (31 - bi).astype(jnp.uint32)))
        cvec = jnp.full((L,), cand)

        def count_body(g, acc):
            for u in range(UNROLL):
                kv = key_v[pl.ds((g * UNROLL + u) * L, L)]
                acc = acc + jnp.where(kv >= cvec, 1.0, 0.0)
            return acc

        cnt = jnp.sum(lax.fori_loop(0, NGU, count_body, zero), axis=0)
        return jnp.where(cnt >= k_f, cand, t)

    t = lax.fori_loop(1, 32, search_body, jnp.uint32(0x80000000))
    tvec = jnp.full((L,), t)

    def tail_body(g, accs):
        cnt_acc, sum_acc = accs
        for u in range(UNROLL):
            kv = key_v[pl.ds((g * UNROLL + u) * L, L)]
            gt = kv > tvec
            cnt_acc = cnt_acc + jnp.where(gt, 1.0, 0.0)
            sum_acc = sum_acc + jnp.where(gt, _key_f32(kv), 0.0)
        return (cnt_acc, sum_acc)

    cnt_gt, sum_gt = lax.fori_loop(0, NGU, tail_body, (zero, zero))
    cnt_gt = jnp.sum(cnt_gt, axis=0)
    sum_gt = jnp.sum(sum_gt, axis=0)
    neg_loss = jnp.where(k > 0, sum_gt + (k_f - cnt_gt) * _key_f32(t), 0.0)

    total = pos_loss + neg_loss + ALPHA * loc_loss
    out_v[...] = jnp.where(ar == 0, total, jnp.where(ar == 1, nm_f, 0.0))
    pltpu.sync_copy(out_v, out_hbm.at[pl.ds(w * L, L)])


def _tc_combine_body(part_ref, o_ref):
    total = jnp.sum(part_ref[:, 0:1], keepdims=True)
    nh = jnp.sum(part_ref[:, 1:2], keepdims=True)
    o_ref[...] = jnp.where(nh == 0.0, jnp.float32(0.0),
                           total / jnp.maximum(nh, 1.0))


def kernel(confidences, localizations, targets):
    pad = NPAD - N
    conf_p = jnp.pad(confidences, ((0, 0), (0, pad), (0, 0)))
    tgt_p = jnp.pad(targets, ((0, 0), (0, pad), (0, 0)))
    loc_p = jnp.pad(localizations, ((0, 0), (0, pad), (0, 0)))
    conf_b = conf_p.reshape(B, NCHUNK, CH, C).transpose(0, 1, 3, 2).reshape(-1)
    tgt_b = tgt_p.reshape(B, NCHUNK, CH, 5).transpose(0, 1, 3, 2).reshape(-1)
    loc_b = loc_p.reshape(B, NCHUNK, CH, 4).transpose(0, 1, 3, 2).reshape(-1)

    mesh = plsc.VectorSubcoreMesh(core_axis_name="c", subcore_axis_name="s",
                                  num_cores=2, num_subcores=16)
    parts = pl.kernel(
        _sc_body,
        out_type=jax.ShapeDtypeStruct((B * L,), jnp.float32),
        mesh=mesh,
        compiler_params=pltpu.CompilerParams(needs_layout_passes=False),
        scratch_types=[
            pltpu.VMEM((CH * C,), jnp.float32),
            pltpu.VMEM((CH * 5,), jnp.float32),
            pltpu.VMEM((CH * 4,), jnp.float32),
            pltpu.VMEM((NPAD,), jnp.uint32),
            pltpu.VMEM((L,), jnp.float32),
            pltpu.SemaphoreType.DMA,
        ],
    )(conf_b, tgt_b, loc_b)

    out = pl.pallas_call(
        _tc_combine_body,
        out_shape=jax.ShapeDtypeStruct((1, 1), jnp.float32),
    )(parts.reshape(B, L))
    return out[0, 0]


# speedup vs baseline: 1.0600x; 1.0029x over previous
import functools

import jax
import jax.numpy as jnp
from jax import lax
from jax.experimental import pallas as pl
from jax.experimental.pallas import tpu as pltpu
from jax.experimental.pallas import tpu_sc as plsc

ALPHA = 1.0
EPS = 1e-7
B = 32
N = 8732
C = 21
L = 16
NPAD = 8736
NCHUNK = 6
CH = NPAD // NCHUNK
GC = CH // L
LN2 = 0.6931471805599453


def _softlog(s):
    bits = lax.bitcast_convert_type(s, jnp.int32)
    e = jnp.float32(1.0) * (lax.shift_right_arithmetic(bits, 23) - 127)
    mbits = lax.bitwise_or(lax.bitwise_and(bits, 0x007FFFFF), 0x3F800000)
    m = lax.bitcast_convert_type(mbits, jnp.float32)
    z = (m - 1.0) / (m + 1.0)
    z2 = z * z
    p = z * (2.0 + z2 * (2.0 / 3.0 + z2 * (2.0 / 5.0 + z2 * (2.0 / 7.0 + z2 * (2.0 / 9.0)))))
    return e * LN2 + p


def _f32_key(v):
    b = lax.bitcast_convert_type(v, jnp.uint32)
    neg = lax.shift_right_logical(b, jnp.uint32(31)) > 0
    return jnp.where(neg, ~b, lax.bitwise_xor(b, jnp.uint32(0x80000000)))


def _key_f32(t):
    was_pos = lax.shift_right_logical(t, jnp.uint32(31)) > 0
    b = jnp.where(was_pos, lax.bitwise_xor(t, jnp.uint32(0x80000000)), ~t)
    return lax.bitcast_convert_type(b, jnp.float32)


def _sc_body(conf_hbm, tgt_hbm, loc_hbm, out_hbm,
             conf_v, tgt_v, loc_v, key_v, out_v, sem):
    w = lax.axis_index("s") * 2 + lax.axis_index("c")
    ar = lax.iota(jnp.int32, L)
    zero = jnp.zeros((L,), jnp.float32)

    def make_group(start_row, masked):
        def group_body(g, accs):
            pos_acc, nm_acc, loc_acc = accs
            base = g * L

            xs = [conf_v[pl.ds(c * CH + base, L)] for c in range(C)]
            m = xs[0]
            for c in range(1, C):
                m = jnp.maximum(m, xs[c])
            s = zero
            for c in range(C):
                s = s + jnp.exp(xs[c] - m)
            lse = m + _softlog(s)

            lab_f = tgt_v[pl.ds(4 * CH + base, L)]
            lab = lab_f.astype(jnp.int32)
            pos = lab > 0
            safe_lab = jnp.where(pos, jnp.clip(lab, 0, C - 1), 0)
            x_lab = plsc.load_gather(conf_v, [safe_lab * CH + base + ar])

            if masked:
                valid = (start_row + base + ar) < N
                posv = jnp.logical_and(pos, valid)
                drop = jnp.logical_or(pos, jnp.logical_not(valid))
            else:
                posv = pos
                drop = pos
            pos_acc = pos_acc + jnp.where(posv, lse - x_lab, 0.0)
            nm_acc = nm_acc + jnp.where(posv, 1.0, 0.0)

            ngv = jnp.where(drop, jnp.float32(-1e30), lse - xs[0])
            key_v[pl.ds(start_row + base, L)] = _f32_key(ngv)

            x1 = loc_v[pl.ds(0 * CH + base, L)]
            y1 = loc_v[pl.ds(1 * CH + base, L)]
            x2 = loc_v[pl.ds(2 * CH + base, L)]
            y2 = loc_v[pl.ds(3 * CH + base, L)]
            x1g = tgt_v[pl.ds(0 * CH + base, L)]
            y1g = tgt_v[pl.ds(1 * CH + base, L)]
            x2g = tgt_v[pl.ds(2 * CH + base, L)]
            y2g = tgt_v[pl.ds(3 * CH + base, L)]
            xkis1 = jnp.maximum(x1, x1g)
            ykis1 = jnp.maximum(y1, y1g)
            xkis2 = jnp.minimum(x2, x2g)
            ykis2 = jnp.minimum(y2, y2g)
            imask = jnp.logical_and(ykis2 > ykis1, xkis2 > xkis1)
            intsctk = jnp.where(imask, (xkis2 - xkis1) * (ykis2 - ykis1), 0.0)
            unionk = (x2 - x1) * (y2 - y1) + (x2g - x1g) * (y2g - y1g) - intsctk
            iouk = intsctk / (unionk + EPS)
            area_c = (jnp.maximum(x2, x2g) - jnp.minimum(x1, x1g)) * \
                     (jnp.maximum(y2, y2g) - jnp.minimum(y1, y1g))
            miouk = iouk - (area_c - unionk) / (area_c + EPS)
            loc_acc = loc_acc + jnp.where(posv, 1.0 - miouk, 0.0)
            return pos_acc, nm_acc, loc_acc

        return group_body

    def chunk_accs(ci, accs, masked):
        c1 = pltpu.async_copy(
            conf_hbm.at[pl.ds((w * NCHUNK + ci) * (CH * C), CH * C)], conf_v, sem)
        c2 = pltpu.async_copy(
            tgt_hbm.at[pl.ds((w * NCHUNK + ci) * (CH * 5), CH * 5)], tgt_v, sem)
        c3 = pltpu.async_copy(
            loc_hbm.at[pl.ds((w * NCHUNK + ci) * (CH * 4), CH * 4)], loc_v, sem)
        c1.wait()
        c2.wait()
        c3.wait()
        return lax.fori_loop(0, GC, make_group(ci * CH, masked), accs)

    accs = lax.fori_loop(
        0, NCHUNK - 1,
        lambda ci, a: chunk_accs(ci, a, False),
        (zero, zero, zero))
    pos_acc, nm_acc, loc_acc = chunk_accs(NCHUNK - 1, accs, True)

    pos_loss = jnp.sum(pos_acc, axis=0)
    nm_f = jnp.sum(nm_acc, axis=0)
    loc_loss = jnp.sum(loc_acc, axis=0)
    nm = nm_f.astype(jnp.int32)
    k = jnp.minimum(3 * nm, N - nm)
    k_f = k.astype(jnp.float32)

    NG = NPAD // L
    UNROLL = 6
    NGU = NG // UNROLL

    def search_body(bi, t):
        cand = lax.bitwise_or(t, lax.shift_left(jnp.uint32(1), (31 - bi).astype(jnp.uint32)))
        cvec = jnp.full((L,), cand)

        def count_body(g, acc):
            for u in range(UNROLL):
                kv = key_v[pl.ds((g * UNROLL + u) * L, L)]
                acc = acc + jnp.where(kv >= cvec, 1.0, 0.0)
            return acc

        cnt = jnp.sum(lax.fori_loop(0, NGU, count_body, zero), axis=0)
        return jnp.where(cnt >= k_f, cand, t)

    t = lax.fori_loop(1, 32, search_body, jnp.uint32(0x80000000))
    tvec = jnp.full((L,), t)

    def tail_body(g, accs):
        cnt_acc, sum_acc = accs
        for u in range(UNROLL):
            kv = key_v[pl.ds((g * UNROLL + u) * L, L)]
            gt = kv > tvec
            cnt_acc = cnt_acc + jnp.where(gt, 1.0, 0.0)
            sum_acc = sum_acc + jnp.where(gt, _key_f32(kv), 0.0)
        return (cnt_acc, sum_acc)

    cnt_gt, sum_gt = lax.fori_loop(0, NGU, tail_body, (zero, zero))
    cnt_gt = jnp.sum(cnt_gt, axis=0)
    sum_gt = jnp.sum(sum_gt, axis=0)
    neg_loss = jnp.where(k > 0, sum_gt + (k_f - cnt_gt) * _key_f32(t), 0.0)

    total = pos_loss + neg_loss + ALPHA * loc_loss
    out_v[...] = jnp.where(ar == 0, total, jnp.where(ar == 1, nm_f, 0.0))
    pltpu.sync_copy(out_v, out_hbm.at[pl.ds(w * L, L)])


def _tc_combine_body(part_ref, o_ref):
    total = jnp.sum(part_ref[:, 0:1], keepdims=True)
    nh = jnp.sum(part_ref[:, 1:2], keepdims=True)
    o_ref[...] = jnp.where(nh == 0.0, jnp.float32(0.0),
                           total / jnp.maximum(nh, 1.0))


def kernel(confidences, localizations, targets):
    pad = NPAD - N
    conf_p = jnp.pad(confidences, ((0, 0), (0, pad), (0, 0)))
    tgt_p = jnp.pad(targets, ((0, 0), (0, pad), (0, 0)))
    loc_p = jnp.pad(localizations, ((0, 0), (0, pad), (0, 0)))
    conf_b = conf_p.reshape(B, NCHUNK, CH, C).transpose(0, 1, 3, 2).reshape(-1)
    tgt_b = tgt_p.reshape(B, NCHUNK, CH, 5).transpose(0, 1, 3, 2).reshape(-1)
    loc_b = loc_p.reshape(B, NCHUNK, CH, 4).transpose(0, 1, 3, 2).reshape(-1)

    mesh = plsc.VectorSubcoreMesh(core_axis_name="c", subcore_axis_name="s",
                                  num_cores=2, num_subcores=16)
    parts = pl.kernel(
        _sc_body,
        out_type=jax.ShapeDtypeStruct((B * L,), jnp.float32),
        mesh=mesh,
        compiler_params=pltpu.CompilerParams(needs_layout_passes=False),
        scratch_types=[
            pltpu.VMEM((CH * C,), jnp.float32),
            pltpu.VMEM((CH * 5,), jnp.float32),
            pltpu.VMEM((CH * 4,), jnp.float32),
            pltpu.VMEM((NPAD,), jnp.uint32),
            pltpu.VMEM((L,), jnp.float32),
            pltpu.SemaphoreType.DMA,
        ],
    )(conf_b, tgt_b, loc_b)

    out = pl.pallas_call(
        _tc_combine_body,
        out_shape=jax.ShapeDtypeStruct((1, 1), jnp.float32),
    )(parts.reshape(B, L))
    return out[0, 0]


# speedup vs baseline: 1.0603x; 1.0003x over previous
import functools

import jax
import jax.numpy as jnp
from jax import lax
from jax.experimental import pallas as pl
from jax.experimental.pallas import tpu as pltpu
from jax.experimental.pallas import tpu_sc as plsc

ALPHA = 1.0
EPS = 1e-7
B = 32
N = 8732
C = 21
L = 16
NPAD = 8736
NCHUNK = 6
CH = NPAD // NCHUNK
GC = CH // L
LN2 = 0.6931471805599453


def _softlog(s):
    bits = lax.bitcast_convert_type(s, jnp.int32)
    e = jnp.float32(1.0) * (lax.shift_right_arithmetic(bits, 23) - 127)
    mbits = lax.bitwise_or(lax.bitwise_and(bits, 0x007FFFFF), 0x3F800000)
    m = lax.bitcast_convert_type(mbits, jnp.float32)
    z = (m - 1.0) / (m + 1.0)
    z2 = z * z
    p = z * (2.0 + z2 * (2.0 / 3.0 + z2 * (2.0 / 5.0 + z2 * (2.0 / 7.0 + z2 * (2.0 / 9.0)))))
    return e * LN2 + p


def _f32_key(v):
    b = lax.bitcast_convert_type(v, jnp.uint32)
    neg = lax.shift_right_logical(b, jnp.uint32(31)) > 0
    return jnp.where(neg, ~b, lax.bitwise_xor(b, jnp.uint32(0x80000000)))


def _key_f32(t):
    was_pos = lax.shift_right_logical(t, jnp.uint32(31)) > 0
    b = jnp.where(was_pos, lax.bitwise_xor(t, jnp.uint32(0x80000000)), ~t)
    return lax.bitcast_convert_type(b, jnp.float32)


def _sc_body(conf_hbm, tgt_hbm, loc_hbm, out_hbm,
             conf_v, tgt_v, loc_v, key_v, out_v, sem):
    w = lax.axis_index("s") * 2 + lax.axis_index("c")
    ar = lax.iota(jnp.int32, L)
    zero = jnp.zeros((L,), jnp.float32)

    def make_group(start_row, masked):
        def group_body(g, accs):
            pos_acc, nm_acc, loc_acc = accs
            base = g * L

            xs = [conf_v[pl.ds(c * CH + base, L)] for c in range(C)]
            m = xs[0]
            for c in range(1, C):
                m = jnp.maximum(m, xs[c])
            s = zero
            for c in range(C):
                s = s + jnp.exp(xs[c] - m)
            lse = m + _softlog(s)

            lab_f = tgt_v[pl.ds(4 * CH + base, L)]
            lab = lab_f.astype(jnp.int32)
            pos = lab > 0
            safe_lab = jnp.where(pos, jnp.clip(lab, 0, C - 1), 0)
            x_lab = plsc.load_gather(conf_v, [safe_lab * CH + base + ar])

            if masked:
                valid = (start_row + base + ar) < N
                posv = jnp.logical_and(pos, valid)
                drop = jnp.logical_or(pos, jnp.logical_not(valid))
            else:
                posv = pos
                drop = pos
            pos_acc = pos_acc + jnp.where(posv, lse - x_lab, 0.0)
            nm_acc = nm_acc + jnp.where(posv, 1.0, 0.0)

            ngv = jnp.where(drop, jnp.float32(-1e30), lse - xs[0])
            key_v[pl.ds(start_row + base, L)] = _f32_key(ngv)

            x1 = loc_v[pl.ds(0 * CH + base, L)]
            y1 = loc_v[pl.ds(1 * CH + base, L)]
            x2 = loc_v[pl.ds(2 * CH + base, L)]
            y2 = loc_v[pl.ds(3 * CH + base, L)]
            x1g = tgt_v[pl.ds(0 * CH + base, L)]
            y1g = tgt_v[pl.ds(1 * CH + base, L)]
            x2g = tgt_v[pl.ds(2 * CH + base, L)]
            y2g = tgt_v[pl.ds(3 * CH + base, L)]
            xkis1 = jnp.maximum(x1, x1g)
            ykis1 = jnp.maximum(y1, y1g)
            xkis2 = jnp.minimum(x2, x2g)
            ykis2 = jnp.minimum(y2, y2g)
            imask = jnp.logical_and(ykis2 > ykis1, xkis2 > xkis1)
            intsctk = jnp.where(imask, (xkis2 - xkis1) * (ykis2 - ykis1), 0.0)
            unionk = (x2 - x1) * (y2 - y1) + (x2g - x1g) * (y2g - y1g) - intsctk
            iouk = intsctk / (unionk + EPS)
            area_c = (jnp.maximum(x2, x2g) - jnp.minimum(x1, x1g)) * \
                     (jnp.maximum(y2, y2g) - jnp.minimum(y1, y1g))
            miouk = iouk - (area_c - unionk) / (area_c + EPS)
            loc_acc = loc_acc + jnp.where(posv, 1.0 - miouk, 0.0)
            return pos_acc, nm_acc, loc_acc

        return group_body

    def chunk_accs(ci, accs, masked):
        c1 = pltpu.async_copy(
            conf_hbm.at[pl.ds((w * NCHUNK + ci) * (CH * C), CH * C)], conf_v, sem)
        c2 = pltpu.async_copy(
            tgt_hbm.at[pl.ds((w * NCHUNK + ci) * (CH * 5), CH * 5)], tgt_v, sem)
        c3 = pltpu.async_copy(
            loc_hbm.at[pl.ds((w * NCHUNK + ci) * (CH * 4), CH * 4)], loc_v, sem)
        c1.wait()
        c2.wait()
        c3.wait()
        return lax.fori_loop(0, GC, make_group(ci * CH, masked), accs)

    accs = lax.fori_loop(
        0, NCHUNK - 1,
        lambda ci, a: chunk_accs(ci, a, False),
        (zero, zero, zero))
    pos_acc, nm_acc, loc_acc = chunk_accs(NCHUNK - 1, accs, True)

    pos_loss = jnp.sum(pos_acc, axis=0)
    nm_f = jnp.sum(nm_acc, axis=0)
    loc_loss = jnp.sum(loc_acc, axis=0)
    nm = nm_f.astype(jnp.int32)
    k = jnp.minimum(3 * nm, N - nm)
    k_f = k.astype(jnp.float32)

    NG = NPAD // L
    UNROLL = 6
    NGU = NG // UNROLL

    def search_body(bi, t):
        cand = lax.bitwise_or(t, lax.shift_left(jnp.uint32(1), (31 - bi).astype(jnp.uint32)))
        cvec = jnp.full((L,), cand)

        @plsc.parallel_loop(0, NG, unroll=UNROLL, carry=zero)
        def count_acc(g, acc):
            kv = key_v[pl.ds(g * L, L)]
            return acc + jnp.where(kv >= cvec, 1.0, 0.0)

        cnt = jnp.sum(count_acc, axis=0)
        return jnp.where(cnt >= k_f, cand, t)

    t = lax.fori_loop(1, 32, search_body, jnp.uint32(0x80000000))
    tvec = jnp.full((L,), t)

    @plsc.parallel_loop(0, NG, unroll=UNROLL, carry=(zero, zero))
    def tail_accs(g, accs):
        cnt_acc, sum_acc = accs
        kv = key_v[pl.ds(g * L, L)]
        gt = kv > tvec
        return (cnt_acc + jnp.where(gt, 1.0, 0.0),
                sum_acc + jnp.where(gt, _key_f32(kv), 0.0))

    cnt_gt, sum_gt = tail_accs
    cnt_gt = jnp.sum(cnt_gt, axis=0)
    sum_gt = jnp.sum(sum_gt, axis=0)
    neg_loss = jnp.where(k > 0, sum_gt + (k_f - cnt_gt) * _key_f32(t), 0.0)

    total = pos_loss + neg_loss + ALPHA * loc_loss
    out_v[...] = jnp.where(ar == 0, total, jnp.where(ar == 1, nm_f, 0.0))
    pltpu.sync_copy(out_v, out_hbm.at[pl.ds(w * L, L)])


def _tc_combine_body(part_ref, o_ref):
    total = jnp.sum(part_ref[:, 0:1], keepdims=True)
    nh = jnp.sum(part_ref[:, 1:2], keepdims=True)
    o_ref[...] = jnp.where(nh == 0.0, jnp.float32(0.0),
                           total / jnp.maximum(nh, 1.0))


def kernel(confidences, localizations, targets):
    pad = NPAD - N
    conf_p = jnp.pad(confidences, ((0, 0), (0, pad), (0, 0)))
    tgt_p = jnp.pad(targets, ((0, 0), (0, pad), (0, 0)))
    loc_p = jnp.pad(localizations, ((0, 0), (0, pad), (0, 0)))
    conf_b = conf_p.reshape(B, NCHUNK, CH, C).transpose(0, 1, 3, 2).reshape(-1)
    tgt_b = tgt_p.reshape(B, NCHUNK, CH, 5).transpose(0, 1, 3, 2).reshape(-1)
    loc_b = loc_p.reshape(B, NCHUNK, CH, 4).transpose(0, 1, 3, 2).reshape(-1)

    mesh = plsc.VectorSubcoreMesh(core_axis_name="c", subcore_axis_name="s",
                                  num_cores=2, num_subcores=16)
    parts = pl.kernel(
        _sc_body,
        out_type=jax.ShapeDtypeStruct((B * L,), jnp.float32),
        mesh=mesh,
        compiler_params=pltpu.CompilerParams(needs_layout_passes=False),
        scratch_types=[
            pltpu.VMEM((CH * C,), jnp.float32),
            pltpu.VMEM((CH * 5,), jnp.float32),
            pltpu.VMEM((CH * 4,), jnp.float32),
            pltpu.VMEM((NPAD,), jnp.uint32),
            pltpu.VMEM((L,), jnp.float32),
            pltpu.SemaphoreType.DMA,
        ],
    )(conf_b, tgt_b, loc_b)

    out = pl.pallas_call(
        _tc_combine_body,
        out_shape=jax.ShapeDtypeStruct((1, 1), jnp.float32),
    )(parts.reshape(B, L))
    return out[0, 0]


# speedup vs baseline: 1.1469x; 1.0817x over previous
import functools

import jax
import jax.numpy as jnp
from jax import lax
from jax.experimental import pallas as pl
from jax.experimental.pallas import tpu as pltpu
from jax.experimental.pallas import tpu_sc as plsc

ALPHA = 1.0
EPS = 1e-7
B = 32
N = 8732
C = 21
L = 16
NPAD = 8736
NCHUNK = 6
CH = NPAD // NCHUNK
GC = CH // L
LN2 = 0.6931471805599453


def _softlog(s):
    bits = lax.bitcast_convert_type(s, jnp.int32)
    e = jnp.float32(1.0) * (lax.shift_right_arithmetic(bits, 23) - 127)
    mbits = lax.bitwise_or(lax.bitwise_and(bits, 0x007FFFFF), 0x3F800000)
    m = lax.bitcast_convert_type(mbits, jnp.float32)
    z = (m - 1.0) / (m + 1.0)
    z2 = z * z
    p = z * (2.0 + z2 * (2.0 / 3.0 + z2 * (2.0 / 5.0 + z2 * (2.0 / 7.0 + z2 * (2.0 / 9.0)))))
    return e * LN2 + p


def _f32_key(v):
    b = lax.bitcast_convert_type(v, jnp.uint32)
    neg = lax.shift_right_logical(b, jnp.uint32(31)) > 0
    return jnp.where(neg, ~b, lax.bitwise_xor(b, jnp.uint32(0x80000000)))


def _key_f32(t):
    was_pos = lax.shift_right_logical(t, jnp.uint32(31)) > 0
    b = jnp.where(was_pos, lax.bitwise_xor(t, jnp.uint32(0x80000000)), ~t)
    return lax.bitcast_convert_type(b, jnp.float32)


def _sc_body(conf_hbm, tgt_hbm, loc_hbm, out_hbm,
             conf_v, tgt_v, loc_v, key_v, out_v, sem):
    w = lax.axis_index("s") * 2 + lax.axis_index("c")
    ar = lax.iota(jnp.int32, L)
    zero = jnp.zeros((L,), jnp.float32)

    def make_group(start_row, masked):
        def group_body(g, accs):
            pos_acc, nm_acc, loc_acc = accs
            base = g * L

            xs = [conf_v[pl.ds(c * CH + base, L)] for c in range(C)]
            m = xs[0]
            for c in range(1, C):
                m = jnp.maximum(m, xs[c])
            s = zero
            for c in range(C):
                s = s + jnp.exp(xs[c] - m)
            lse = m + _softlog(s)

            lab_f = tgt_v[pl.ds(4 * CH + base, L)]
            lab = lab_f.astype(jnp.int32)
            pos = lab > 0
            safe_lab = jnp.where(pos, jnp.clip(lab, 0, C - 1), 0)
            x_lab = plsc.load_gather(conf_v, [safe_lab * CH + base + ar])

            if masked:
                valid = (start_row + base + ar) < N
                posv = jnp.logical_and(pos, valid)
                drop = jnp.logical_or(pos, jnp.logical_not(valid))
            else:
                posv = pos
                drop = pos
            pos_acc = pos_acc + jnp.where(posv, lse - x_lab, 0.0)
            nm_acc = nm_acc + jnp.where(posv, 1.0, 0.0)

            ngv = jnp.where(drop, jnp.float32(-1e30), lse - xs[0])
            key_v[pl.ds(start_row + base, L)] = _f32_key(ngv)

            x1 = loc_v[pl.ds(0 * CH + base, L)]
            y1 = loc_v[pl.ds(1 * CH + base, L)]
            x2 = loc_v[pl.ds(2 * CH + base, L)]
            y2 = loc_v[pl.ds(3 * CH + base, L)]
            x1g = tgt_v[pl.ds(0 * CH + base, L)]
            y1g = tgt_v[pl.ds(1 * CH + base, L)]
            x2g = tgt_v[pl.ds(2 * CH + base, L)]
            y2g = tgt_v[pl.ds(3 * CH + base, L)]
            xkis1 = jnp.maximum(x1, x1g)
            ykis1 = jnp.maximum(y1, y1g)
            xkis2 = jnp.minimum(x2, x2g)
            ykis2 = jnp.minimum(y2, y2g)
            imask = jnp.logical_and(ykis2 > ykis1, xkis2 > xkis1)
            intsctk = jnp.where(imask, (xkis2 - xkis1) * (ykis2 - ykis1), 0.0)
            unionk = (x2 - x1) * (y2 - y1) + (x2g - x1g) * (y2g - y1g) - intsctk
            iouk = intsctk / (unionk + EPS)
            area_c = (jnp.maximum(x2, x2g) - jnp.minimum(x1, x1g)) * \
                     (jnp.maximum(y2, y2g) - jnp.minimum(y1, y1g))
            miouk = iouk - (area_c - unionk) / (area_c + EPS)
            loc_acc = loc_acc + jnp.where(posv, 1.0 - miouk, 0.0)
            return pos_acc, nm_acc, loc_acc

        return group_body

    def chunk_accs(ci, accs, masked):
        c1 = pltpu.async_copy(
            conf_hbm.at[pl.ds((w * NCHUNK + ci) * (CH * C), CH * C)], conf_v, sem)
        c2 = pltpu.async_copy(
            tgt_hbm.at[pl.ds((w * NCHUNK + ci) * (CH * 5), CH * 5)], tgt_v, sem)
        c3 = pltpu.async_copy(
            loc_hbm.at[pl.ds((w * NCHUNK + ci) * (CH * 4), CH * 4)], loc_v, sem)
        c1.wait()
        c2.wait()
        c3.wait()
        body = make_group(ci * CH, masked)

        @plsc.parallel_loop(0, GC, unroll=2, carry=accs)
        def chunk_out(g, a):
            return body(g, a)

        return chunk_out

    accs = lax.fori_loop(
        0, NCHUNK - 1,
        lambda ci, a: chunk_accs(ci, a, False),
        (zero, zero, zero))
    pos_acc, nm_acc, loc_acc = chunk_accs(NCHUNK - 1, accs, True)

    pos_loss = jnp.sum(pos_acc, axis=0)
    nm_f = jnp.sum(nm_acc, axis=0)
    loc_loss = jnp.sum(loc_acc, axis=0)
    nm = nm_f.astype(jnp.int32)
    k = jnp.minimum(3 * nm, N - nm)
    k_f = k.astype(jnp.float32)

    NG = NPAD // L
    UNROLL = 6
    NGU = NG // UNROLL

    def search_body(bi, t):
        cand = lax.bitwise_or(t, lax.shift_left(jnp.uint32(1), (31 - bi).astype(jnp.uint32)))
        cvec = jnp.full((L,), cand)

        @plsc.parallel_loop(0, NG, unroll=UNROLL, carry=zero)
        def count_acc(g, acc):
            kv = key_v[pl.ds(g * L, L)]
            return acc + jnp.where(kv >= cvec, 1.0, 0.0)

        cnt = jnp.sum(count_acc, axis=0)
        return jnp.where(cnt >= k_f, cand, t)

    t = lax.fori_loop(1, 32, search_body, jnp.uint32(0x80000000))
    tvec = jnp.full((L,), t)

    @plsc.parallel_loop(0, NG, unroll=UNROLL, carry=(zero, zero))
    def tail_accs(g, accs):
        cnt_acc, sum_acc = accs
        kv = key_v[pl.ds(g * L, L)]
        gt = kv > tvec
        return (cnt_acc + jnp.where(gt, 1.0, 0.0),
                sum_acc + jnp.where(gt, _key_f32(kv), 0.0))

    cnt_gt, sum_gt = tail_accs
    cnt_gt = jnp.sum(cnt_gt, axis=0)
    sum_gt = jnp.sum(sum_gt, axis=0)
    neg_loss = jnp.where(k > 0, sum_gt + (k_f - cnt_gt) * _key_f32(t), 0.0)

    total = pos_loss + neg_loss + ALPHA * loc_loss
    out_v[...] = jnp.where(ar == 0, total, jnp.where(ar == 1, nm_f, 0.0))
    pltpu.sync_copy(out_v, out_hbm.at[pl.ds(w * L, L)])


def _tc_combine_body(part_ref, o_ref):
    total = jnp.sum(part_ref[:, 0:1], keepdims=True)
    nh = jnp.sum(part_ref[:, 1:2], keepdims=True)
    o_ref[...] = jnp.where(nh == 0.0, jnp.float32(0.0),
                           total / jnp.maximum(nh, 1.0))


def kernel(confidences, localizations, targets):
    pad = NPAD - N
    conf_p = jnp.pad(confidences, ((0, 0), (0, pad), (0, 0)))
    tgt_p = jnp.pad(targets, ((0, 0), (0, pad), (0, 0)))
    loc_p = jnp.pad(localizations, ((0, 0), (0, pad), (0, 0)))
    conf_b = conf_p.reshape(B, NCHUNK, CH, C).transpose(0, 1, 3, 2).reshape(-1)
    tgt_b = tgt_p.reshape(B, NCHUNK, CH, 5).transpose(0, 1, 3, 2).reshape(-1)
    loc_b = loc_p.reshape(B, NCHUNK, CH, 4).transpose(0, 1, 3, 2).reshape(-1)

    mesh = plsc.VectorSubcoreMesh(core_axis_name="c", subcore_axis_name="s",
                                  num_cores=2, num_subcores=16)
    parts = pl.kernel(
        _sc_body,
        out_type=jax.ShapeDtypeStruct((B * L,), jnp.float32),
        mesh=mesh,
        compiler_params=pltpu.CompilerParams(needs_layout_passes=False),
        scratch_types=[
            pltpu.VMEM((CH * C,), jnp.float32),
            pltpu.VMEM((CH * 5,), jnp.float32),
            pltpu.VMEM((CH * 4,), jnp.float32),
            pltpu.VMEM((NPAD,), jnp.uint32),
            pltpu.VMEM((L,), jnp.float32),
            pltpu.SemaphoreType.DMA,
        ],
    )(conf_b, tgt_b, loc_b)

    out = pl.pallas_call(
        _tc_combine_body,
        out_shape=jax.ShapeDtypeStruct((1, 1), jnp.float32),
    )(parts.reshape(B, L))
    return out[0, 0]
